# Initial kernel scaffold; baseline (speedup 1.0000x reference)
import jax
import jax.numpy as jnp
from jax.experimental import pallas as pl


def kernel(hidden_states, type_ids, W, b):
    raise NotImplementedError("write your pallas kernel here")


# speedup vs baseline: 1.4147x; 1.4147x over previous
import functools

import jax
import jax.numpy as jnp
from jax import lax
from jax.experimental import pallas as pl
from jax.experimental.pallas import tpu as pltpu
from jax.experimental.pallas import tpu_sc as plsc

E = 4
D = 2048
R = 256
T = 4096
MAXB = T // R + E
CAP = MAXB * R

TROWS = 32
TLANES = 128

NW = 32
TOK_PER_W = T // NW
CH = 32
NCH = TOK_PER_W // CH


def _routing_body(tid_ref, p_ref, gid_ref):
    tid = tid_ref[...]
    rk = lax.broadcasted_iota(jnp.int32, (TLANES, TLANES), 0)
    ck = lax.broadcasted_iota(jnp.int32, (TLANES, TLANES), 1)
    upper_incl = (rk <= ck).astype(jnp.float32)
    rr = lax.broadcasted_iota(jnp.int32, (TROWS, TROWS), 0)
    cr = lax.broadcasted_iota(jnp.int32, (TROWS, TROWS), 1)
    lower_strict = (cr < rr).astype(jnp.float32)

    ranks = []
    counts = []
    for g in range(E):
        m = (tid == g).astype(jnp.float32)
        lane_cum = jnp.dot(m, upper_incl, preferred_element_type=jnp.float32,
                           precision=lax.Precision.HIGHEST)
        row_tot = jnp.sum(m, axis=1, keepdims=True)
        row_excl = jnp.dot(lower_strict, row_tot,
                           preferred_element_type=jnp.float32,
                           precision=lax.Precision.HIGHEST)
        ranks.append(row_excl + lane_cum - 1.0)
        counts.append(jnp.sum(m))

    p = jnp.zeros((TROWS, TLANES), jnp.float32)
    base = jnp.float32(0.0)
    nblk_cum = []
    acc = jnp.float32(0.0)
    for g in range(E):
        p = jnp.where(tid == g, base + ranks[g], p)
        nblk = jnp.ceil(counts[g] / R)
        base = base + nblk * R
        acc = acc + nblk
        nblk_cum.append(acc)
    p_ref[...] = p.astype(jnp.int32)

    ii = lax.broadcasted_iota(jnp.int32, (1, TLANES), 1).astype(jnp.float32)
    gid = jnp.zeros((1, TLANES), jnp.float32)
    for g in range(E):
        gid = gid + (ii >= nblk_cum[g]).astype(jnp.float32)
    gid_ref[...] = jnp.minimum(gid, float(E - 1)).astype(jnp.int32)


_routing = pl.pallas_call(
    _routing_body,
    out_shape=(
        jax.ShapeDtypeStruct((TROWS, TLANES), jnp.int32),
        jax.ShapeDtypeStruct((1, TLANES), jnp.int32),
    ),
)


def _mm_body(gid_ref, x_ref, w_ref, b_ref, y_ref):
    del gid_ref
    y_ref[...] = (
        jnp.dot(x_ref[...], w_ref[0], preferred_element_type=jnp.float32)
        + b_ref[0]
    )


_grouped_mm = pl.pallas_call(
    _mm_body,
    grid_spec=pltpu.PrefetchScalarGridSpec(
        num_scalar_prefetch=1,
        grid=(MAXB,),
        in_specs=[
            pl.BlockSpec((R, D), lambda i, gid: (i, 0)),
            pl.BlockSpec((1, D, D), lambda i, gid: (gid[i], 0, 0)),
            pl.BlockSpec((1, 1, D), lambda i, gid: (gid[i], 0, 0)),
        ],
        out_specs=pl.BlockSpec((R, D), lambda i, gid: (i, 0)),
    ),
    out_shape=jax.ShapeDtypeStruct((CAP, D), jnp.float32),
)

@functools.cache
def _sc_kernels():
    mesh = plsc.VectorSubcoreMesh(core_axis_name="c", subcore_axis_name="s")

    @functools.partial(
        pl.kernel,
        out_type=jax.ShapeDtypeStruct((CAP, D), jnp.float32),
        mesh=mesh,
        scratch_types=[
            pltpu.VMEM((CH,), jnp.int32),
            pltpu.VMEM((CH, D), jnp.float32),
            pltpu.SemaphoreType.DMA,
        ],
    )
    def sc_scatter(h_hbm, p_hbm, xs_hbm, idx_v, rows_v, sem):
        wid = lax.axis_index("s") * 2 + lax.axis_index("c")
        base = wid * TOK_PER_W
        for c in range(NCH):
            off = base + c * CH
            pltpu.sync_copy(p_hbm.at[pl.ds(off, CH)], idx_v)
            pltpu.sync_copy(h_hbm.at[pl.ds(off, CH)], rows_v)
            pltpu.async_copy(rows_v, xs_hbm.at[idx_v], sem).wait()

    @functools.partial(
        pl.kernel,
        out_type=jax.ShapeDtypeStruct((T, D), jnp.float32),
        mesh=mesh,
        scratch_types=[
            pltpu.VMEM((CH,), jnp.int32),
            pltpu.VMEM((CH, D), jnp.float32),
            pltpu.SemaphoreType.DMA,
        ],
    )
    def sc_gather(y_hbm, p_hbm, out_hbm, idx_v, rows_v, sem):
        wid = lax.axis_index("s") * 2 + lax.axis_index("c")
        base = wid * TOK_PER_W
        for c in range(NCH):
            off = base + c * CH
            pltpu.sync_copy(p_hbm.at[pl.ds(off, CH)], idx_v)
            pltpu.async_copy(y_hbm.at[idx_v], rows_v, sem).wait()
            pltpu.sync_copy(rows_v, out_hbm.at[pl.ds(off, CH)])

    return sc_scatter, sc_gather


@jax.jit
def kernel(hidden_states, type_ids, W, b):
    B, S, _ = hidden_states.shape
    h2d = hidden_states.reshape(T, D)
    tid = type_ids.reshape(TROWS, TLANES).astype(jnp.int32)
    p2d, gid_row = _routing(tid)
    p = p2d.reshape(T)
    block_gid = gid_row[0, :MAXB]
    sc_scatter, sc_gather = _sc_kernels()
    x_sorted = sc_scatter(h2d, p)
    y_sorted = _grouped_mm(block_gid, x_sorted, W, b.reshape(E, 1, D))
    out = sc_gather(y_sorted, p)
    return out.reshape(B, S, D)


# speedup vs baseline: 1.4443x; 1.0209x over previous
import functools

import jax
import jax.numpy as jnp
from jax import lax
from jax.experimental import pallas as pl
from jax.experimental.pallas import tpu as pltpu
from jax.experimental.pallas import tpu_sc as plsc

E = 4
D = 2048
R = 256
T = 4096
MAXB = T // R + E
CAP = MAXB * R

TROWS = 32
TLANES = 128

NW = 32
TOK_PER_W = T // NW
CH = 16
NCH = TOK_PER_W // CH


def _routing_body(tid_ref, p_ref, gid_ref):
    tid = tid_ref[...]
    rk = lax.broadcasted_iota(jnp.int32, (TLANES, TLANES), 0)
    ck = lax.broadcasted_iota(jnp.int32, (TLANES, TLANES), 1)
    upper_incl = (rk <= ck).astype(jnp.float32)
    rr = lax.broadcasted_iota(jnp.int32, (TROWS, TROWS), 0)
    cr = lax.broadcasted_iota(jnp.int32, (TROWS, TROWS), 1)
    lower_strict = (cr < rr).astype(jnp.float32)

    ranks = []
    counts = []
    for g in range(E):
        m = (tid == g).astype(jnp.float32)
        lane_cum = jnp.dot(m, upper_incl, preferred_element_type=jnp.float32,
                           precision=lax.Precision.HIGHEST)
        row_tot = jnp.sum(m, axis=1, keepdims=True)
        row_excl = jnp.dot(lower_strict, row_tot,
                           preferred_element_type=jnp.float32,
                           precision=lax.Precision.HIGHEST)
        ranks.append(row_excl + lane_cum - 1.0)
        counts.append(jnp.sum(m))

    p = jnp.zeros((TROWS, TLANES), jnp.float32)
    base = jnp.float32(0.0)
    nblk_cum = []
    acc = jnp.float32(0.0)
    for g in range(E):
        p = jnp.where(tid == g, base + ranks[g], p)
        nblk = jnp.ceil(counts[g] / R)
        base = base + nblk * R
        acc = acc + nblk
        nblk_cum.append(acc)
    p_ref[...] = p.astype(jnp.int32)

    ii = lax.broadcasted_iota(jnp.int32, (1, TLANES), 1).astype(jnp.float32)
    gid = jnp.zeros((1, TLANES), jnp.float32)
    for g in range(E):
        gid = gid + (ii >= nblk_cum[g]).astype(jnp.float32)
    gid = jnp.minimum(gid, float(E - 1))
    gid = jnp.where(ii == float(MAXB), nblk_cum[E - 1], gid)
    gid_ref[...] = gid.astype(jnp.int32)


_routing = pl.pallas_call(
    _routing_body,
    out_shape=(
        jax.ShapeDtypeStruct((TROWS, TLANES), jnp.int32),
        jax.ShapeDtypeStruct((1, TLANES), jnp.int32),
    ),
)


def _mm_body(gid_ref, x_ref, w_ref, b_ref, y_ref):
    @pl.when(pl.program_id(0) < gid_ref[MAXB])
    def _():
        y_ref[...] = (
            jnp.dot(x_ref[...], w_ref[0], preferred_element_type=jnp.float32)
            + b_ref[0]
        )


_grouped_mm = pl.pallas_call(
    _mm_body,
    grid_spec=pltpu.PrefetchScalarGridSpec(
        num_scalar_prefetch=1,
        grid=(MAXB,),
        in_specs=[
            pl.BlockSpec((R, D), lambda i, gid: (i, 0)),
            pl.BlockSpec((1, D, D), lambda i, gid: (gid[i], 0, 0)),
            pl.BlockSpec((1, 1, D), lambda i, gid: (gid[i], 0, 0)),
        ],
        out_specs=pl.BlockSpec((R, D), lambda i, gid: (i, 0)),
    ),
    out_shape=jax.ShapeDtypeStruct((CAP, D), jnp.float32),
)

@functools.cache
def _sc_kernels():
    mesh = plsc.VectorSubcoreMesh(core_axis_name="c", subcore_axis_name="s")
    scratch = [
        pltpu.VMEM((NCH, CH), jnp.int32),
        pltpu.VMEM((CH, D), jnp.float32),
        pltpu.VMEM((CH, D), jnp.float32),
        pltpu.SemaphoreType.DMA,
        pltpu.SemaphoreType.DMA,
        pltpu.SemaphoreType.DMA,
    ]

    @functools.partial(
        pl.kernel,
        out_type=jax.ShapeDtypeStruct((CAP, D), jnp.float32),
        mesh=mesh,
        scratch_types=scratch,
    )
    def sc_scatter(h_hbm, p_hbm, xs_hbm, idx_v, buf0, buf1, lsem0, lsem1, ssem):
        wid = lax.axis_index("s") * 2 + lax.axis_index("c")
        base = wid * TOK_PER_W
        bufs = (buf0, buf1)
        lsems = (lsem0, lsem1)
        pltpu.sync_copy(p_hbm.at[pl.ds(wid * NCH, NCH)], idx_v)
        pltpu.async_copy(h_hbm.at[pl.ds(base, CH)], buf0, lsem0)
        pltpu.async_copy(h_hbm.at[pl.ds(base + CH, CH)], buf1, lsem1)
        for c in range(NCH):
            b = c % 2
            pltpu.make_async_copy(
                h_hbm.at[pl.ds(base, CH)], bufs[b], lsems[b]
            ).wait()
            pltpu.async_copy(bufs[b], xs_hbm.at[idx_v.at[c]], ssem).wait()
            if c + 2 < NCH:
                pltpu.async_copy(
                    h_hbm.at[pl.ds(base + (c + 2) * CH, CH)], bufs[b], lsems[b]
                )

    @functools.partial(
        pl.kernel,
        out_type=jax.ShapeDtypeStruct((T, D), jnp.float32),
        mesh=mesh,
        scratch_types=scratch,
    )
    def sc_gather(y_hbm, p_hbm, out_hbm, idx_v, buf0, buf1, gsem0, gsem1, wsem):
        wid = lax.axis_index("s") * 2 + lax.axis_index("c")
        base = wid * TOK_PER_W
        bufs = (buf0, buf1)
        gsems = (gsem0, gsem1)
        pltpu.sync_copy(p_hbm.at[pl.ds(wid * NCH, NCH)], idx_v)
        pltpu.async_copy(y_hbm.at[idx_v.at[0]], buf0, gsem0)
        pltpu.async_copy(y_hbm.at[idx_v.at[1]], buf1, gsem1)
        for c in range(NCH):
            b = c % 2
            pltpu.make_async_copy(
                y_hbm.at[idx_v.at[c]], bufs[b], gsems[b]
            ).wait()
            pltpu.async_copy(
                bufs[b], out_hbm.at[pl.ds(base + c * CH, CH)], wsem
            ).wait()
            if c + 2 < NCH:
                pltpu.async_copy(y_hbm.at[idx_v.at[c + 2]], bufs[b], gsems[b])

    return sc_scatter, sc_gather


@jax.jit
def kernel(hidden_states, type_ids, W, b):
    B, S, _ = hidden_states.shape
    h2d = hidden_states.reshape(T, D)
    tid = type_ids.reshape(TROWS, TLANES).astype(jnp.int32)
    p2d, gid_row = _routing(tid)
    p_chunks = p2d.reshape(T // CH, CH)
    block_gid = gid_row[0, : MAXB + 1]
    sc_scatter, sc_gather = _sc_kernels()
    x_sorted = sc_scatter(h2d, p_chunks)
    y_sorted = _grouped_mm(block_gid, x_sorted, W, b.reshape(E, 1, D))
    out = sc_gather(y_sorted, p_chunks)
    return out.reshape(B, S, D)
